# baseline (speedup 1.0000x reference)
import functools

import jax
import jax.numpy as jnp
from jax import lax
from jax.experimental import pallas as pl
from jax.experimental.pallas import tpu as pltpu
from jax.experimental.pallas import tpu_sc as plsc

B = 64
NC = 2
NS = 16
CPS = B // NS
NPAIR = CPS // 2
CHUNKI = 4096
LANES = 16
GUNROLL = 8
XSHIFT = 14


def _sc_body(x_hbm, idxp_hbm, vals_hbm, out_hbm,
             a0, a1, acc0, acc1, acc2, acc3,
             bi0, bv0, bi1, bv1, t0, t1, sem0, sem1):
  n = a0.shape[0]
  n_chunks = idxp_hbm.shape[0]
  seg = t0.shape[0]
  cid = lax.axis_index("c")
  sid = lax.axis_index("s")

  accs = [acc0, acc1, acc2, acc3]
  pairs = [a0, a1]
  bufs = [(bi0, bv0, sem0), (bi1, bv1, sem1)]

  for k in range(NPAIR):
    r0 = sid * CPS + 2 * k
    for j in range(n // seg):
      pltpu.sync_copy(x_hbm.at[r0].at[pl.ds(j * seg, seg)], t0)
      pltpu.sync_copy(x_hbm.at[r0 + 1].at[pl.ds(j * seg, seg)], t1)

      @plsc.parallel_loop(0, seg // LANES, unroll=4)
      def _pack(g):
        sl = pl.ds(g * LANES, LANES)
        pw = plsc.bitcast(
            plsc.pack(t0[sl], t1[sl], format=plsc.PackFormat.INTERLEAVED),
            jnp.float32,
        )
        pairs[k][pl.ds(j * seg + g * LANES, LANES)] = pw

  zero16 = jnp.zeros((LANES,), jnp.float32)

  @plsc.parallel_loop(0, n // LANES)
  def _zero(i):
    for acc in accs:
      acc[pl.ds(i * LANES, LANES)] = zero16

  def issue(chunk, b):
    bi, bv, sem = bufs[b]
    pltpu.async_copy(idxp_hbm.at[chunk].at[cid], bi, sem)
    pltpu.async_copy(vals_hbm.at[chunk].at[cid], bv, sem)

  def wait(chunk, b):
    bi, bv, sem = bufs[b]
    pltpu.make_async_copy(idxp_hbm.at[chunk].at[cid], bi, sem).wait()
    pltpu.make_async_copy(vals_hbm.at[chunk].at[cid], bv, sem).wait()

  issue(0, 0)
  issue(1, 1)

  himask = jnp.full((LANES,), -65536, jnp.int32)
  xmask = jnp.full((LANES,), (1 << XSHIFT) - 1, jnp.int32)

  @pl.loop(0, n_chunks, step=2)
  def _main(h):
    for b in range(2):
      cc = h + b
      bi, bv, _ = bufs[b]
      wait(cc, b)

      @plsc.parallel_loop(0, CHUNKI // LANES, unroll=GUNROLL)
      def _group(g):
        sl = pl.ds(g * LANES, LANES)
        vp = bi[sl]
        vv = bv[sl]
        vx = vp & xmask
        vy = lax.shift_right_logical(vp, XSHIFT)
        for k in range(NPAIR):
          gp = plsc.load_gather(pairs[k], [vx])
          gi = plsc.bitcast(gp, jnp.int32)
          xe = plsc.bitcast(gi << 16, jnp.float32)
          xo = plsc.bitcast(gi & himask, jnp.float32)
          plsc.addupdate_scatter(accs[2 * k], [vy], xe * vv)
          plsc.addupdate_scatter(accs[2 * k + 1], [vy], xo * vv)

      @pl.when(cc + 2 < n_chunks)
      def _refill():
        issue(cc + 2, b)

  for k in range(CPS):
    pltpu.sync_copy(accs[k], out_hbm.at[cid].at[sid * CPS + k])


def _make_sc_spmm(n, n_chunks):
  mesh = plsc.VectorSubcoreMesh(core_axis_name="c", subcore_axis_name="s")
  return pl.kernel(
      _sc_body,
      out_type=jax.ShapeDtypeStruct((NC, B, n), jnp.float32),
      mesh=mesh,
      scratch_types=[pltpu.VMEM((n,), jnp.float32) for _ in range(2 + CPS)]
      + [
          pltpu.VMEM((CHUNKI,), jnp.int32),
          pltpu.VMEM((CHUNKI,), jnp.float32),
          pltpu.VMEM((CHUNKI,), jnp.int32),
          pltpu.VMEM((CHUNKI,), jnp.float32),
          pltpu.VMEM((4096,), jnp.float32),
          pltpu.VMEM((4096,), jnp.float32),
          pltpu.SemaphoreType.DMA,
          pltpu.SemaphoreType.DMA,
      ],
      compiler_params=pltpu.CompilerParams(
          use_tc_tiling_on_sc=True, needs_layout_passes=False
      ),
  )


def _combine_out(parts):
  n = parts.shape[-1]
  blk = 2048

  def body(p_ref, o_ref):
    o_ref[...] = p_ref[0] + p_ref[1]

  return pl.pallas_call(
      body,
      grid=(n // blk,),
      in_specs=[pl.BlockSpec((NC, B, blk), lambda j: (0, 0, j))],
      out_specs=pl.BlockSpec((B, blk), lambda j: (0, j)),
      out_shape=jax.ShapeDtypeStruct((B, n), jnp.float32),
  )(parts)


@jax.jit
def kernel(x, vals, idx_xs, idx_ys):
  n = x.shape[1]
  nnz = vals.shape[0]
  per_round = NC * CHUNKI
  n_chunks = -(-nnz // per_round)
  if n_chunks % 2:
    n_chunks += 1
  items = n_chunks * per_round
  pad = items - nnz

  idxp = (idx_ys << XSHIFT) | idx_xs
  idxp = jnp.concatenate([idxp, jnp.zeros((pad,), jnp.int32)])
  v = jnp.concatenate([vals, jnp.zeros((pad,), jnp.float32)])
  idxp = idxp.reshape(n_chunks, NC, CHUNKI)
  v = v.reshape(n_chunks, NC, CHUNKI)

  parts = _make_sc_spmm(n, n_chunks)(x, idxp, v)
  return _combine_out(parts)


# speedup vs baseline: 1.1499x; 1.1499x over previous
import functools

import jax
import jax.numpy as jnp
from jax import lax
from jax.experimental import pallas as pl
from jax.experimental.pallas import tpu as pltpu
from jax.experimental.pallas import tpu_sc as plsc

B = 64
NC = 2
NS = 16
CPS = B // NS
NPAIR = CPS // 2
CHUNKI = 2048
LANES = 16
GUNROLL = 4
XSHIFT = 14


def _sc_body(x_hbm, idxp_hbm, vals_hbm, out_hbm,
             a0, a1, acc0, acc1, acc2, acc3,
             bi0, bv0, bi1, bv1, t0, t1, sem0, sem1):
  n = a0.shape[0]
  n_chunks = idxp_hbm.shape[0]
  seg = t0.shape[0]
  cid = lax.axis_index("c")
  sid = lax.axis_index("s")

  accs = [acc0, acc1, acc2, acc3]
  pairs = [a0, a1]
  bufs = [(bi0, bv0, sem0), (bi1, bv1, sem1)]

  for k in range(NPAIR):
    r0 = sid * CPS + 2 * k
    for j in range(n // seg):
      pltpu.sync_copy(x_hbm.at[r0].at[pl.ds(j * seg, seg)], t0)
      pltpu.sync_copy(x_hbm.at[r0 + 1].at[pl.ds(j * seg, seg)], t1)

      @plsc.parallel_loop(0, seg // LANES, unroll=4)
      def _pack(g):
        sl = pl.ds(g * LANES, LANES)
        pw = plsc.bitcast(
            plsc.pack(t0[sl], t1[sl], format=plsc.PackFormat.INTERLEAVED),
            jnp.float32,
        )
        pairs[k][pl.ds(j * seg + g * LANES, LANES)] = pw

  zero16 = jnp.zeros((LANES,), jnp.float32)

  @plsc.parallel_loop(0, n // LANES)
  def _zero(i):
    for acc in accs:
      acc[pl.ds(i * LANES, LANES)] = zero16

  def issue(chunk, b):
    bi, bv, sem = bufs[b]
    pltpu.async_copy(idxp_hbm.at[chunk].at[cid], bi, sem)
    pltpu.async_copy(vals_hbm.at[chunk].at[cid], bv, sem)

  def wait(chunk, b):
    bi, bv, sem = bufs[b]
    pltpu.make_async_copy(idxp_hbm.at[chunk].at[cid], bi, sem).wait()
    pltpu.make_async_copy(vals_hbm.at[chunk].at[cid], bv, sem).wait()

  issue(0, 0)
  issue(1, 1)

  himask = jnp.full((LANES,), -65536, jnp.int32)
  xmask = jnp.full((LANES,), (1 << XSHIFT) - 1, jnp.int32)

  @pl.loop(0, n_chunks, step=2)
  def _main(h):
    for b in range(2):
      cc = h + b
      bi, bv, _ = bufs[b]
      wait(cc, b)

      @plsc.parallel_loop(0, CHUNKI // LANES, unroll=GUNROLL)
      def _group(g):
        sl = pl.ds(g * LANES, LANES)
        vp = bi[sl]
        vv = bv[sl]
        vx = vp & xmask
        vy = lax.shift_right_logical(vp, XSHIFT)
        for k in range(NPAIR):
          gp = plsc.load_gather(pairs[k], [vx])
          gi = plsc.bitcast(gp, jnp.int32)
          xe = plsc.bitcast(gi << 16, jnp.float32)
          xo = plsc.bitcast(gi & himask, jnp.float32)
          plsc.addupdate_scatter(accs[2 * k], [vy], xe * vv)
          plsc.addupdate_scatter(accs[2 * k + 1], [vy], xo * vv)

      @pl.when(cc + 2 < n_chunks)
      def _refill():
        issue(cc + 2, b)

  for k in range(CPS):
    pltpu.sync_copy(accs[k], out_hbm.at[cid].at[sid * CPS + k])


def _make_sc_spmm(n, n_chunks):
  mesh = plsc.VectorSubcoreMesh(core_axis_name="c", subcore_axis_name="s")
  return pl.kernel(
      _sc_body,
      out_type=jax.ShapeDtypeStruct((NC, B, n), jnp.float32),
      mesh=mesh,
      scratch_types=[pltpu.VMEM((n,), jnp.float32) for _ in range(2 + CPS)]
      + [
          pltpu.VMEM((CHUNKI,), jnp.int32),
          pltpu.VMEM((CHUNKI,), jnp.float32),
          pltpu.VMEM((CHUNKI,), jnp.int32),
          pltpu.VMEM((CHUNKI,), jnp.float32),
          pltpu.VMEM((4096,), jnp.float32),
          pltpu.VMEM((4096,), jnp.float32),
          pltpu.SemaphoreType.DMA,
          pltpu.SemaphoreType.DMA,
      ],
      compiler_params=pltpu.CompilerParams(
          use_tc_tiling_on_sc=True, needs_layout_passes=False
      ),
  )


def _combine_out(parts):
  n = parts.shape[-1]
  blk = 2048

  def body(p_ref, o_ref):
    o_ref[...] = p_ref[0] + p_ref[1]

  return pl.pallas_call(
      body,
      grid=(n // blk,),
      in_specs=[pl.BlockSpec((NC, B, blk), lambda j: (0, 0, j))],
      out_specs=pl.BlockSpec((B, blk), lambda j: (0, j)),
      out_shape=jax.ShapeDtypeStruct((B, n), jnp.float32),
  )(parts)


@jax.jit
def kernel(x, vals, idx_xs, idx_ys):
  n = x.shape[1]
  nnz = vals.shape[0]
  per_round = NC * CHUNKI
  n_chunks = -(-nnz // per_round)
  if n_chunks % 2:
    n_chunks += 1
  items = n_chunks * per_round
  pad = items - nnz

  idxp = (idx_ys << XSHIFT) | idx_xs
  idxp = jnp.concatenate([idxp, jnp.zeros((pad,), jnp.int32)])
  v = jnp.concatenate([vals, jnp.zeros((pad,), jnp.float32)])
  idxp = idxp.reshape(n_chunks, NC, CHUNKI)
  v = v.reshape(n_chunks, NC, CHUNKI)

  parts = _make_sc_spmm(n, n_chunks)(x, idxp, v)
  return _combine_out(parts)
